# baseline (speedup 1.0000x reference)
import jax
import jax.numpy as jnp
from jax.experimental import pallas as pl
from jax.experimental.pallas import tpu as pltpu

D = 768
P, Q, S = 7, 1, 69
N = 5
T = P + Q + S
CHUNKS = ((0, 24), (24, 24), (48, 21))


def _concat_body(p_ref, q_ref, s_ref, o_ref, v_ref, sem_in, sem_out):
    ip = pltpu.make_async_copy(p_ref, v_ref.at[:, :P, :], sem_in.at[0])
    iq = pltpu.make_async_copy(q_ref, v_ref.at[:, P, :], sem_in.at[1])
    i_s = [
        pltpu.make_async_copy(
            s_ref.at[:, off : off + sz, :],
            v_ref.at[:, P + Q + off : P + Q + off + sz, :],
            sem_in.at[2 + k],
        )
        for k, (off, sz) in enumerate(CHUNKS)
    ]
    for c in i_s:
        c.start()
    ip.start()
    iq.start()

    ip.wait()
    iq.wait()
    o0 = pltpu.make_async_copy(
        v_ref.at[:, : P + Q, :], o_ref.at[:, : P + Q, :], sem_out.at[0]
    )
    o0.start()
    outs = [o0]
    for k, (off, sz) in enumerate(CHUNKS):
        i_s[k].wait()
        ok = pltpu.make_async_copy(
            v_ref.at[:, P + Q + off : P + Q + off + sz, :],
            o_ref.at[:, P + Q + off : P + Q + off + sz, :],
            sem_out.at[1 + k],
        )
        ok.start()
        outs.append(ok)
    for c in outs:
        c.wait()


def kernel(embedding_prefix, learnable_quality, embedding_suffix):
    return pl.pallas_call(
        _concat_body,
        out_shape=jax.ShapeDtypeStruct((N, T, D), jnp.float32),
        in_specs=[
            pl.BlockSpec(memory_space=pl.ANY),
            pl.BlockSpec(memory_space=pl.ANY),
            pl.BlockSpec(memory_space=pl.ANY),
        ],
        out_specs=pl.BlockSpec(memory_space=pl.ANY),
        scratch_shapes=[
            pltpu.VMEM((N, T, D), jnp.float32),
            pltpu.SemaphoreType.DMA((2 + len(CHUNKS),)),
            pltpu.SemaphoreType.DMA((1 + len(CHUNKS),)),
        ],
    )(embedding_prefix, learnable_quality, embedding_suffix)


# speedup vs baseline: 1.0647x; 1.0647x over previous
import jax
import jax.numpy as jnp
from jax.experimental import pallas as pl
from jax.experimental.pallas import tpu as pltpu

D = 768
P, Q, S = 7, 1, 69
N = 5
T = P + Q + S
CHUNKS = ((0, 16), (16, 16), (32, 16), (48, 21))


def _concat_body(p_ref, q_ref, s_ref, o_ref, v_ref, sem_in, sem_out):
    ip = pltpu.make_async_copy(p_ref, v_ref.at[:, :P, :], sem_in.at[0])
    iq = pltpu.make_async_copy(q_ref, v_ref.at[:, P, :], sem_in.at[1])
    i_s = [
        pltpu.make_async_copy(
            s_ref.at[:, off : off + sz, :],
            v_ref.at[:, P + Q + off : P + Q + off + sz, :],
            sem_in.at[2 + k],
        )
        for k, (off, sz) in enumerate(CHUNKS)
    ]
    ip.start()
    iq.start()
    for c in i_s:
        c.start()

    ip.wait()
    iq.wait()
    o0 = pltpu.make_async_copy(
        v_ref.at[:, : P + Q, :], o_ref.at[:, : P + Q, :], sem_out.at[0]
    )
    o0.start()
    outs = [o0]
    for k, (off, sz) in enumerate(CHUNKS):
        i_s[k].wait()
        ok = pltpu.make_async_copy(
            v_ref.at[:, P + Q + off : P + Q + off + sz, :],
            o_ref.at[:, P + Q + off : P + Q + off + sz, :],
            sem_out.at[1 + k],
        )
        ok.start()
        outs.append(ok)
    for c in outs:
        c.wait()


def kernel(embedding_prefix, learnable_quality, embedding_suffix):
    return pl.pallas_call(
        _concat_body,
        out_shape=jax.ShapeDtypeStruct((N, T, D), jnp.float32),
        in_specs=[
            pl.BlockSpec(memory_space=pl.ANY),
            pl.BlockSpec(memory_space=pl.ANY),
            pl.BlockSpec(memory_space=pl.ANY),
        ],
        out_specs=pl.BlockSpec(memory_space=pl.ANY),
        scratch_shapes=[
            pltpu.VMEM((N, T, D), jnp.float32),
            pltpu.SemaphoreType.DMA((2 + len(CHUNKS),)),
            pltpu.SemaphoreType.DMA((1 + len(CHUNKS),)),
        ],
    )(embedding_prefix, learnable_quality, embedding_suffix)


# speedup vs baseline: 1.0730x; 1.0078x over previous
import jax
import jax.numpy as jnp
from jax.experimental import pallas as pl
from jax.experimental.pallas import tpu as pltpu

D = 768
P, Q, S = 7, 1, 69
N = 5
T = P + Q + S
CHUNKS = ((0, 24), (24, 24), (48, 21))


def _concat_body(p_ref, q_ref, s_ref, o_ref, v_ref, sem_in, sem_out):
    ip = pltpu.make_async_copy(p_ref, v_ref.at[:, :P, :], sem_in.at[0])
    iq = pltpu.make_async_copy(q_ref, v_ref.at[:, P, :], sem_in.at[1])
    i_s = [
        pltpu.make_async_copy(
            s_ref.at[:, off : off + sz, :],
            v_ref.at[:, P + Q + off : P + Q + off + sz, :],
            sem_in.at[2 + k],
        )
        for k, (off, sz) in enumerate(CHUNKS)
    ]
    ip.start()
    iq.start()
    for c in i_s:
        c.start()

    ip.wait()
    iq.wait()
    o0 = pltpu.make_async_copy(
        v_ref.at[:, : P + Q, :], o_ref.at[:, : P + Q, :], sem_out.at[0]
    )
    o0.start()
    outs = [o0]
    for k, (off, sz) in enumerate(CHUNKS):
        i_s[k].wait()
        ok = pltpu.make_async_copy(
            v_ref.at[:, P + Q + off : P + Q + off + sz, :],
            o_ref.at[:, P + Q + off : P + Q + off + sz, :],
            sem_out.at[1 + k],
        )
        ok.start()
        outs.append(ok)
    for c in outs:
        c.wait()


def kernel(embedding_prefix, learnable_quality, embedding_suffix):
    return pl.pallas_call(
        _concat_body,
        out_shape=jax.ShapeDtypeStruct((N, T, D), jnp.float32),
        in_specs=[
            pl.BlockSpec(memory_space=pl.ANY),
            pl.BlockSpec(memory_space=pl.ANY),
            pl.BlockSpec(memory_space=pl.ANY),
        ],
        out_specs=pl.BlockSpec(memory_space=pl.ANY),
        scratch_shapes=[
            pltpu.VMEM((N, T, D), jnp.float32),
            pltpu.SemaphoreType.DMA((2 + len(CHUNKS),)),
            pltpu.SemaphoreType.DMA((1 + len(CHUNKS),)),
        ],
    )(embedding_prefix, learnable_quality, embedding_suffix)
